# baseline (speedup 1.0000x reference)
import functools

import jax
import jax.numpy as jnp
from jax.experimental import pallas as pl
from jax.experimental.pallas import tpu as pltpu


def _gat_body(x_ref, wfct_ref, acomb_ref, bgat_ref, wproj_ref, bproj_ref,
              out_ref, *, H, D):
    HD = H * D
    xb = x_ref[0]
    wfct = wfct_ref[...]
    acomb = acomb_ref[...]

    f32 = jnp.float32
    feat = jax.lax.dot_general(xb, wfct, (((0,), (0,)), ((), ())),
                               preferred_element_type=f32)
    lr = jax.lax.dot_general(feat, acomb, (((1,), (0,)), ((), ())),
                             preferred_element_type=f32)
    lrT = jax.lax.dot_general(acomb, feat, (((0,), (1,)), ((), ())),
                              preferred_element_type=f32)

    rst_parts = []
    for h in range(H):
        el_col = lr[:, h:h + 1]
        er_row = lrT[H + h:H + h + 1, :]
        e = el_col + er_row
        e = jnp.where(e >= 0, e, 0.2 * e)
        m = jnp.max(e, axis=0, keepdims=True)
        p = jnp.exp(e - m)
        s = jnp.sum(p, axis=0, keepdims=True)
        alpha = p / s
        feat_h = feat[:, h * D:(h + 1) * D]
        rst_parts.append(
            jax.lax.dot_general(alpha, feat_h, (((0,), (0,)), ((), ())),
                                preferred_element_type=f32))

    rst = jnp.concatenate(rst_parts, axis=1) + bgat_ref[...]
    outT = jax.lax.dot_general(wproj_ref[...], rst, (((1,), (1,)), ((), ())),
                               preferred_element_type=f32)
    out_ref[0] = outT + bproj_ref[...]


def kernel(x, W_fc, attn_l, attn_r, bias_gat, W_proj, b_proj):
    B, W, F = x.shape
    H, D = attn_l.shape
    HD = H * D

    f32 = jnp.float32
    Wfc_T = W_fc.astype(f32).T
    eye = jnp.eye(H, dtype=f32)
    Al = (attn_l.astype(f32)[:, :, None] * eye[:, None, :]).reshape(HD, H)
    Ar = (attn_r.astype(f32)[:, :, None] * eye[:, None, :]).reshape(HD, H)
    A_comb = jnp.concatenate([Al, Ar], axis=1)
    bgat = bias_gat.astype(f32).reshape(1, HD)
    bproj = b_proj.astype(f32).reshape(W, 1)

    body = functools.partial(_gat_body, H=H, D=D)
    out = pl.pallas_call(
        body,
        grid=(B,),
        in_specs=[
            pl.BlockSpec((1, W, F), lambda b: (b, 0, 0)),
            pl.BlockSpec((W, HD), lambda b: (0, 0)),
            pl.BlockSpec((HD, 2 * H), lambda b: (0, 0)),
            pl.BlockSpec((1, HD), lambda b: (0, 0)),
            pl.BlockSpec((W, HD), lambda b: (0, 0)),
            pl.BlockSpec((W, 1), lambda b: (0, 0)),
        ],
        out_specs=pl.BlockSpec((1, W, F), lambda b: (b, 0, 0)),
        out_shape=jax.ShapeDtypeStruct((B, W, F), f32),
        compiler_params=pltpu.CompilerParams(
            dimension_semantics=("parallel",)),
    )(x.astype(f32), Wfc_T, A_comb, bgat, W_proj.astype(f32), bproj)
    return out
